# baseline (speedup 1.0000x reference)
import functools
import math

import jax
import jax.numpy as jnp
from jax.experimental import pallas as pl


def _score_body(q_ref, s_ref, qb_ref):
    q = q_ref[...]
    s_ref[...] = jnp.sum(q * q, axis=1)
    qb_ref[...] = q.astype(jnp.bfloat16)


def _select_body(u, s_ref, p_ref):
    s2 = s_ref[...]
    G, T = s2.shape
    L = 128
    R = T // L
    s_int = jax.lax.bitcast_convert_type(s2, jnp.int32)

    def bs1(i, t):
        cand = t | (jnp.int32(1) << (30 - i))
        w = jnp.where(s_int >= cand, 1, 0)
        cnt = jnp.sum(w, axis=1, keepdims=True)
        return jnp.where(cnt >= u, cand, t)

    thr = jax.lax.fori_loop(0, 31, bs1, jnp.zeros((G, 1), jnp.int32))

    gt = s_int > thr
    eq = s_int == thr
    c_gt = jnp.sum(jnp.where(gt, 1, 0), axis=1, keepdims=True)
    need = u - c_gt

    idx2 = jax.lax.broadcasted_iota(jnp.int32, (G, T), 1)
    nbits = max(1, (T - 1).bit_length())

    def bs2(i, m):
        cand = m | (jnp.int32(1) << (nbits - 1 - i))
        w = jnp.where(eq & (idx2 < cand), 1, 0)
        f = jnp.sum(w, axis=1, keepdims=True)
        return jnp.where(f < need, cand, m)

    M = jax.lax.fori_loop(0, nbits, bs2, jnp.zeros((G, 1), jnp.int32))

    sel = gt | (eq & (idx2 <= M))
    self32 = sel.astype(jnp.float32)

    upperL = (jax.lax.broadcasted_iota(jnp.int32, (L, L), 0)
              < jax.lax.broadcasted_iota(jnp.int32, (L, L), 1)
              ).astype(jnp.float32)
    base = jnp.zeros((G, 1), jnp.float32)
    for j in range(R):
        blk = self32[:, j * L:(j + 1) * L]
        row_in = jnp.dot(blk, upperL, preferred_element_type=jnp.float32)
        pos = (row_in + base).astype(jnp.int32)
        p_ref[:, j, 0, :] = jnp.where(sel[:, j * L:(j + 1) * L], pos, u)
        base = base + jnp.sum(blk, axis=1, keepdims=True)


def _attn_body(u, qb_ref, k_ref, v_ref, p_ref, o_ref):
    qtb = qb_ref[0]
    kt = k_ref[0]
    vt = v_ref[0]
    posp = p_ref[0, :, 0, :]
    R = posp.shape[0]
    Dh = qtb.shape[0]

    iu = jax.lax.broadcasted_iota(jnp.int32, (128, 128), 0)
    chunks = [iu == posp[r:r + 1, :] for r in range(R)]
    P = jnp.concatenate(chunks, axis=1).astype(jnp.bfloat16)

    ktb = kt.astype(jnp.bfloat16)
    vtb = vt.astype(jnp.bfloat16)

    q_selt = jax.lax.dot_general(qtb, P, (((1,), (1,)), ((), ())),
                                 preferred_element_type=jnp.float32)
    scores = jax.lax.dot_general(q_selt.astype(jnp.bfloat16), ktb,
                                 (((0,), (0,)), ((), ())),
                                 preferred_element_type=jnp.float32)
    scores = scores * (1.0 / math.sqrt(Dh))
    mx = jnp.max(scores, axis=1, keepdims=True)
    e = jnp.exp(scores - mx)
    den = jnp.sum(e, axis=1, keepdims=True)
    o_unnt = jax.lax.dot_general(vtb, e.astype(jnp.bfloat16),
                                 (((1,), (1,)), ((), ())),
                                 preferred_element_type=jnp.float32)

    eye = (jax.lax.broadcasted_iota(jnp.int32, (128, 128), 0)
           == jax.lax.broadcasted_iota(jnp.int32, (128, 128), 1))
    diag = eye.astype(jnp.float32) * (1.0 / den)
    o_selt = jnp.dot(o_unnt, diag, preferred_element_type=jnp.float32)

    meanvt = jnp.mean(vt, axis=1, keepdims=True)
    ir = jax.lax.broadcasted_iota(jnp.int32, (1, 128), 1)
    o_delt = jnp.where(ir == u, 0.0, o_selt - meanvt)
    scat = jax.lax.dot_general(o_delt.astype(jnp.bfloat16), P,
                               (((1,), (0,)), ((), ())),
                               preferred_element_type=jnp.float32)
    o_ref[0] = scat + meanvt


def kernel(Q, K, V):
    B, H, Tq, Dh = Q.shape
    Tk = K.shape[2]
    u = max(1, min(Tq, int(math.ceil(math.log(Tk + 1) * 16))))
    G = B * H
    R = Tq // 128
    Qt = jnp.swapaxes(Q, -1, -2).reshape(G, Dh, Tq)
    Kt = jnp.swapaxes(K, -1, -2).reshape(G, Dh, Tk)
    Vt = jnp.swapaxes(V, -1, -2).reshape(G, Dh, Tk)

    s2, Qtb = pl.pallas_call(
        _score_body,
        grid=(R,),
        in_specs=[pl.BlockSpec((G, Dh, 128), lambda j: (0, 0, j))],
        out_specs=[
            pl.BlockSpec((G, 128), lambda j: (0, j)),
            pl.BlockSpec((G, Dh, 128), lambda j: (0, 0, j)),
        ],
        out_shape=[
            jax.ShapeDtypeStruct((G, Tq), jnp.float32),
            jax.ShapeDtypeStruct((G, Dh, Tq), jnp.bfloat16),
        ],
    )(Qt)

    posp = pl.pallas_call(
        functools.partial(_select_body, u),
        out_shape=jax.ShapeDtypeStruct((G, R, 1, 128), jnp.int32),
    )(s2)

    outt = pl.pallas_call(
        functools.partial(_attn_body, u),
        grid=(G,),
        in_specs=[
            pl.BlockSpec((1, Dh, Tq), lambda i: (i, 0, 0)),
            pl.BlockSpec((1, Dh, Tk), lambda i: (i, 0, 0)),
            pl.BlockSpec((1, Dh, Tk), lambda i: (i, 0, 0)),
            pl.BlockSpec((1, R, 1, 128), lambda i: (i, 0, 0, 0)),
        ],
        out_specs=pl.BlockSpec((1, Dh, Tq), lambda i: (i, 0, 0)),
        out_shape=jax.ShapeDtypeStruct((G, Dh, Tq), jnp.float32),
    )(Qtb, Kt, Vt, posp)
    return jnp.swapaxes(outt.reshape(B, H, Dh, Tq), -1, -2)


# speedup vs baseline: 1.0545x; 1.0545x over previous
import functools
import math

import jax
import jax.numpy as jnp
from jax.experimental import pallas as pl


def _score_body(q_ref, s_ref, qb_ref):
    q = q_ref[...]
    s_ref[...] = jnp.sum(q * q, axis=1)
    qb_ref[...] = q.astype(jnp.bfloat16)


def _select_body(u, s_ref, p_ref):
    s2 = s_ref[...]
    G, T = s2.shape
    L = 128
    R = T // L
    s_int = jax.lax.bitcast_convert_type(s2, jnp.int32)

    def bs1(i, t):
        cand = t | (jnp.int32(1) << (30 - i))
        w = jnp.where(s_int >= cand, 1, 0)
        cnt = jnp.sum(w, axis=1, keepdims=True)
        return jnp.where(cnt >= u, cand, t)

    thr = jax.lax.fori_loop(0, 31, bs1, jnp.zeros((G, 1), jnp.int32))

    gt = s_int > thr
    eq = s_int == thr
    c_gt = jnp.sum(jnp.where(gt, 1, 0), axis=1, keepdims=True)
    need = u - c_gt

    idx2 = jax.lax.broadcasted_iota(jnp.int32, (G, T), 1)
    nbits = max(1, (T - 1).bit_length())

    def bs2(i, m):
        cand = m | (jnp.int32(1) << (nbits - 1 - i))
        w = jnp.where(eq & (idx2 < cand), 1, 0)
        f = jnp.sum(w, axis=1, keepdims=True)
        return jnp.where(f < need, cand, m)

    M = jax.lax.fori_loop(0, nbits, bs2, jnp.zeros((G, 1), jnp.int32))

    sel = gt | (eq & (idx2 <= M))
    self32 = sel.astype(jnp.float32)

    upperL = (jax.lax.broadcasted_iota(jnp.int32, (L, L), 0)
              < jax.lax.broadcasted_iota(jnp.int32, (L, L), 1)
              ).astype(jnp.float32)
    base = jnp.zeros((G, 1), jnp.float32)
    for j in range(R):
        blk = self32[:, j * L:(j + 1) * L]
        row_in = jnp.dot(blk, upperL, preferred_element_type=jnp.float32)
        pos = (row_in + base).astype(jnp.int32)
        p_ref[:, j, 0, :] = jnp.where(sel[:, j * L:(j + 1) * L], pos, u)
        base = base + jnp.sum(blk, axis=1, keepdims=True)


def _attn_body(u, qb_ref, k_ref, v_ref, p_ref, o_ref):
    qtb = qb_ref[0]
    kt = k_ref[0]
    vt = v_ref[0]
    posp = p_ref[0, :, 0, :]
    R = posp.shape[0]
    Dh = qtb.shape[0]

    iu = jax.lax.broadcasted_iota(jnp.int32, (128, 128), 0)
    chunks = [iu == posp[r:r + 1, :] for r in range(R)]
    P = jnp.concatenate(chunks, axis=1).astype(jnp.bfloat16)

    ktb = kt.astype(jnp.bfloat16)
    vtb = vt.astype(jnp.bfloat16)

    q_selt = jax.lax.dot_general(qtb, P, (((1,), (1,)), ((), ())),
                                 preferred_element_type=jnp.float32)
    scores = jax.lax.dot_general(q_selt.astype(jnp.bfloat16), ktb,
                                 (((0,), (0,)), ((), ())),
                                 preferred_element_type=jnp.float32)
    scores = scores * (1.0 / math.sqrt(Dh))
    mx = jnp.max(scores, axis=1, keepdims=True)
    e = jnp.exp(scores - mx)
    v_aug = jnp.concatenate(
        [vtb, jnp.ones((1, vtb.shape[1]), jnp.bfloat16)], axis=0)
    o_aug = jax.lax.dot_general(v_aug, e.astype(jnp.bfloat16),
                                (((1,), (1,)), ((), ())),
                                preferred_element_type=jnp.float32)
    o_selt = o_aug[0:Dh, :] * (1.0 / o_aug[Dh:Dh + 1, :])

    meanvt = jnp.mean(vt, axis=1, keepdims=True)
    ir = jax.lax.broadcasted_iota(jnp.int32, (1, 128), 1)
    o_delt = jnp.where(ir == u, 0.0, o_selt - meanvt)
    scat = jax.lax.dot_general(o_delt.astype(jnp.bfloat16), P,
                               (((1,), (0,)), ((), ())),
                               preferred_element_type=jnp.float32)
    o_ref[0] = scat + meanvt


def kernel(Q, K, V):
    B, H, Tq, Dh = Q.shape
    Tk = K.shape[2]
    u = max(1, min(Tq, int(math.ceil(math.log(Tk + 1) * 16))))
    G = B * H
    R = Tq // 128
    Qt = jnp.swapaxes(Q, -1, -2).reshape(G, Dh, Tq)
    Kt = jnp.swapaxes(K, -1, -2).reshape(G, Dh, Tk)
    Vt = jnp.swapaxes(V, -1, -2).reshape(G, Dh, Tk)

    s2, Qtb = pl.pallas_call(
        _score_body,
        grid=(R,),
        in_specs=[pl.BlockSpec((G, Dh, 128), lambda j: (0, 0, j))],
        out_specs=[
            pl.BlockSpec((G, 128), lambda j: (0, j)),
            pl.BlockSpec((G, Dh, 128), lambda j: (0, 0, j)),
        ],
        out_shape=[
            jax.ShapeDtypeStruct((G, Tq), jnp.float32),
            jax.ShapeDtypeStruct((G, Dh, Tq), jnp.bfloat16),
        ],
    )(Qt)

    posp = pl.pallas_call(
        functools.partial(_select_body, u),
        out_shape=jax.ShapeDtypeStruct((G, R, 1, 128), jnp.int32),
    )(s2)

    outt = pl.pallas_call(
        functools.partial(_attn_body, u),
        grid=(G,),
        in_specs=[
            pl.BlockSpec((1, Dh, Tq), lambda i: (i, 0, 0)),
            pl.BlockSpec((1, Dh, Tk), lambda i: (i, 0, 0)),
            pl.BlockSpec((1, Dh, Tk), lambda i: (i, 0, 0)),
            pl.BlockSpec((1, R, 1, 128), lambda i: (i, 0, 0, 0)),
        ],
        out_specs=pl.BlockSpec((1, Dh, Tq), lambda i: (i, 0, 0)),
        out_shape=jax.ShapeDtypeStruct((G, Dh, Tq), jnp.float32),
    )(Qtb, Kt, Vt, posp)
    return jnp.swapaxes(outt.reshape(B, H, Dh, Tq), -1, -2)
